# baseline (speedup 1.0000x reference)
import functools

import jax
import jax.numpy as jnp
from jax import lax
from jax.experimental import pallas as pl
from jax.experimental.pallas import tpu as pltpu
from jax.experimental.pallas import tpu_sc as plsc

E = 128

_NC, _NS = 2, 16
_NW = _NC * _NS

_CH = 128
_RB = 800
_UB = 128


def _precompute_body(im_ref, imf_ref, w0i_ref, b0_ref, c_ref):
    t = jnp.dot(im_ref[:], w0i_ref[:], preferred_element_type=jnp.float32)
    t = t + b0_ref[:]
    tb = (lax.bitcast_convert_type(t, jnp.uint32) + jnp.uint32(0x8000)) \
        & jnp.uint32(0xFFFF0000)
    mb = (lax.bitcast_convert_type(imf_ref[:], jnp.uint32)
          + jnp.uint32(0x8000)) >> 16
    c_ref[:] = lax.bitcast_convert_type(tb | mb, jnp.int32)


def _build_c(item_mlp, item_mf, w0i, b0):
    i_rows = item_mlp.shape[0]
    grid = i_rows // _RB
    return pl.pallas_call(
        _precompute_body,
        grid=(grid,),
        in_specs=[
            pl.BlockSpec((_RB, E), lambda i: (i, 0)),
            pl.BlockSpec((_RB, E), lambda i: (i, 0)),
            pl.BlockSpec((E, E), lambda i: (0, 0)),
            pl.BlockSpec((1, E), lambda i: (0, 0)),
        ],
        out_specs=pl.BlockSpec((_RB, E), lambda i: (i, 0)),
        out_shape=jax.ShapeDtypeStruct((i_rows, E), jnp.int32),
    )(item_mlp, item_mf, w0i, b0.reshape(1, E))


def _sc_user_body(upw, uid_hbm, umlp_hbm, umf_hbm, uemlp_out, uemf_out,
                  uidb, urows, semu):
    wid = lax.axis_index("s") * _NC + lax.axis_index("c")
    ubase = wid * upw
    pltpu.sync_copy(uid_hbm.at[pl.ds(ubase, upw)], uidb)
    pltpu.async_copy(umlp_hbm.at[uidb], urows, semu).wait()
    pltpu.sync_copy(urows, uemlp_out.at[pl.ds(ubase, upw)])
    pltpu.async_copy(umf_hbm.at[uidb], urows, semu).wait()
    pltpu.sync_copy(urows, uemf_out.at[pl.ds(ubase, upw)])


def _sc_user_gather(uid, user_mlp, user_mf):
    b = uid.shape[0]
    upw = b // _NW
    mesh = plsc.VectorSubcoreMesh(core_axis_name="c", subcore_axis_name="s")
    return pl.kernel(
        functools.partial(_sc_user_body, upw),
        out_type=[
            jax.ShapeDtypeStruct((b, E), jnp.float32),
            jax.ShapeDtypeStruct((b, E), jnp.float32),
        ],
        mesh=mesh,
        scratch_types=[
            pltpu.VMEM((upw,), jnp.int32),
            pltpu.VMEM((upw, E), jnp.float32),
            pltpu.SemaphoreType.DMA,
        ],
    )(uid, user_mlp, user_mf)


def _sc_token_body(tok, nch,
                   idx_hbm, c_hbm, g_out,
                   idxall, rows0, rows1,
                   semg0, semg1, sems0, sems1):
    wid = lax.axis_index("s") * _NC + lax.axis_index("c")
    tbase = wid * (tok // _NW)
    pltpu.sync_copy(idx_hbm.at[wid], idxall)
    bufs = ((rows0, semg0, sems0), (rows1, semg1, sems1))

    def g_copy(c, rbuf, sem):
        return pltpu.make_async_copy(c_hbm.at[idxall.at[c]], rbuf, sem)

    def s_copy(c, rbuf, sem):
        dst = g_out.at[pl.ds(tbase + c * _CH, _CH)]
        return pltpu.make_async_copy(rbuf, dst, sem)

    g_copy(0, rows0, semg0).start()

    def body(i, carry):
        cbase = i * 2
        for bsel in range(2):
            c = cbase + bsel
            rcur, gcur, scur = bufs[bsel]
            rnxt, gnxt, snxt = bufs[1 - bsel]

            @pl.when(c + 1 < nch)
            def _():
                @pl.when(c >= 1)
                def _():
                    s_copy(c - 1, rnxt, snxt).wait()

                g_copy(c + 1, rnxt, gnxt).start()

            g_copy(c, rcur, gcur).wait()
            s_copy(c, rcur, scur).start()
        return carry

    lax.fori_loop(0, nch // 2, body, 0)
    if nch % 2 == 1:
        rcur, gcur, scur = bufs[(nch - 1) % 2]
        g_copy(nch - 1, rcur, gcur).wait()
        s_copy(nch - 1, rcur, scur).start()
    s_copy(nch - 2, bufs[(nch - 2) % 2][0], bufs[(nch - 2) % 2][2]).wait()
    s_copy(nch - 1, bufs[(nch - 1) % 2][0], bufs[(nch - 1) % 2][2]).wait()


def _sc_token_gather(idx3, c):
    nw, nch, ch = idx3.shape
    tok = nw * nch * ch
    mesh = plsc.VectorSubcoreMesh(core_axis_name="c", subcore_axis_name="s")
    return pl.kernel(
        functools.partial(_sc_token_body, tok, nch),
        out_type=jax.ShapeDtypeStruct((tok, E), jnp.int32),
        mesh=mesh,
        scratch_types=[
            pltpu.VMEM((nch, _CH), jnp.int32),
            pltpu.VMEM((_CH, E), jnp.int32),
            pltpu.VMEM((_CH, E), jnp.int32),
            pltpu.SemaphoreType.DMA,
            pltpu.SemaphoreType.DMA,
            pltpu.SemaphoreType.DMA,
            pltpu.SemaphoreType.DMA,
        ],
    )(idx3, c)


def _mlp_body(tb, g_ref, ue_ref, uemf_ref, w0u_ref, w1_ref, b1_ref,
              w2_ref, b2_ref, woutr_ref, bout_ref, out_ref):
    f32 = jnp.float32
    per_u = tb // _UB
    rowu = lax.broadcasted_iota(jnp.int32, (tb, _UB), 0) // per_u
    colu = lax.broadcasted_iota(jnp.int32, (tb, _UB), 1)
    eb = (rowu == colu).astype(f32)

    a = jnp.dot(ue_ref[:], w0u_ref[:], preferred_element_type=f32)
    up = uemf_ref[:] * woutr_ref[1:2, :]
    a_tok = jnp.dot(eb, a, preferred_element_type=f32)
    up_tok = jnp.dot(eb, up, preferred_element_type=f32)

    gu = lax.bitcast_convert_type(g_ref[:], jnp.uint32)
    gt = lax.bitcast_convert_type(gu & jnp.uint32(0xFFFF0000), f32)
    gm = lax.bitcast_convert_type(gu << 16, f32)
    h = jnp.maximum(gt + a_tok, 0.0)
    h = jnp.maximum(
        jnp.dot(h.astype(jnp.bfloat16), w1_ref[:],
                preferred_element_type=f32) + b1_ref[:], 0.0)
    h = jnp.maximum(
        jnp.dot(h.astype(jnp.bfloat16), w2_ref[:],
                preferred_element_type=f32) + b2_ref[:], 0.0)
    dn = (((1,), (1,)), ((), ()))
    lh = lax.dot_general(woutr_ref[0:1, :], h, dn,
                         preferred_element_type=f32)
    ones_row = jnp.ones((1, E), f32)
    lmf = lax.dot_general(ones_row, gm * up_tok, dn,
                          preferred_element_type=f32)
    out_ref[:] = (lh + lmf + bout_ref[0, 0])[None]


def _mlp(g, uemlp, uemf, w0u, w1, b1, w2, b2, woutr, bout, nusers, ublk0):
    tok = g.shape[0]
    tb = tok // (nusers // _UB)
    grid = nusers // _UB
    return pl.pallas_call(
        functools.partial(_mlp_body, tb),
        grid=(grid,),
        in_specs=[
            pl.BlockSpec((tb, E), lambda i: (i, 0)),
            pl.BlockSpec((_UB, E), lambda i, u0=ublk0: (u0 + i, 0)),
            pl.BlockSpec((_UB, E), lambda i, u0=ublk0: (u0 + i, 0)),
            pl.BlockSpec((E, E), lambda i: (0, 0)),
            pl.BlockSpec((E, E), lambda i: (0, 0)),
            pl.BlockSpec((1, E), lambda i: (0, 0)),
            pl.BlockSpec((E, E), lambda i: (0, 0)),
            pl.BlockSpec((1, E), lambda i: (0, 0)),
            pl.BlockSpec((2, E), lambda i: (0, 0)),
            pl.BlockSpec((1, 1), lambda i: (0, 0)),
        ],
        out_specs=pl.BlockSpec((1, 1, tb), lambda i: (i, 0, 0)),
        out_shape=jax.ShapeDtypeStruct((grid, 1, tb), jnp.float32),
    )(g, uemlp, uemf, w0u, w1, b1, w2, b2, woutr, bout)


def kernel(uid, seq, pos, neg, nbr, nbr_iid, user_mlp, item_mlp, user_mf,
           item_mf, W0, b0, W1, b1, W2, b2, Wout, bout):
    del seq, nbr, nbr_iid
    b_sz, l_sz = pos.shape

    w0u = W0[:E, :]
    w0i = W0[E:, :]
    woutr = Wout.reshape(2, E)

    uemlp, uemf = _sc_user_gather(uid.astype(jnp.int32), user_mlp, user_mf)
    c = _build_c(item_mlp, item_mf, w0i, b0)

    nphase = 2
    tok = 2 * b_sz * l_sz
    tokp = tok // nphase
    nch = tokp // (_NW * _CH)
    all_idx = jnp.concatenate([pos, neg], axis=1).reshape(-1).astype(jnp.int32)
    idx4 = all_idx.reshape(nphase, _NW, nch, _CH)

    w1b = W1.astype(jnp.bfloat16)
    w2b = W2.astype(jnp.bfloat16)
    nusers_p = b_sz // nphase
    parts = []
    for p in range(nphase):
        g_p = _sc_token_gather(idx4[p], c)
        parts.append(_mlp(g_p, uemlp, uemf, w0u, w1b, b1.reshape(1, E),
                          w2b, b2.reshape(1, E), woutr, bout.reshape(1, 1),
                          nusers_p, p * (nusers_p // _UB)))

    logits = jnp.concatenate(parts, axis=0)
    out2 = logits.reshape(b_sz, 2 * l_sz)
    pos_logits = out2[:, :l_sz, None]
    neg_logits = out2[:, l_sz:, None]
    return (pos_logits, neg_logits)


# speedup vs baseline: 1.0158x; 1.0158x over previous
import functools

import jax
import jax.numpy as jnp
from jax import lax
from jax.experimental import pallas as pl
from jax.experimental.pallas import tpu as pltpu
from jax.experimental.pallas import tpu_sc as plsc

E = 128

_NC, _NS = 2, 16
_NW = _NC * _NS

_CH = 128
_RB = 800
_UB = 128


def _precompute_body(im_ref, imf_ref, w0i_ref, b0_ref, c_ref):
    t = jnp.dot(im_ref[:], w0i_ref[:], preferred_element_type=jnp.float32)
    t = t + b0_ref[:]
    tb = (lax.bitcast_convert_type(t, jnp.uint32) + jnp.uint32(0x8000)) \
        & jnp.uint32(0xFFFF0000)
    mb = (lax.bitcast_convert_type(imf_ref[:], jnp.uint32)
          + jnp.uint32(0x8000)) >> 16
    c_ref[:] = lax.bitcast_convert_type(tb | mb, jnp.int32)


def _build_c(item_mlp, item_mf, w0i, b0):
    i_rows = item_mlp.shape[0]
    grid = i_rows // _RB
    return pl.pallas_call(
        _precompute_body,
        grid=(grid,),
        in_specs=[
            pl.BlockSpec((_RB, E), lambda i: (i, 0)),
            pl.BlockSpec((_RB, E), lambda i: (i, 0)),
            pl.BlockSpec((E, E), lambda i: (0, 0)),
            pl.BlockSpec((1, E), lambda i: (0, 0)),
        ],
        out_specs=pl.BlockSpec((_RB, E), lambda i: (i, 0)),
        out_shape=jax.ShapeDtypeStruct((i_rows, E), jnp.int32),
    )(item_mlp, item_mf, w0i, b0.reshape(1, E))


def _sc_user_body(upw, uid_hbm, umlp_hbm, umf_hbm, uemlp_out, uemf_out,
                  uidb, urows, semu):
    wid = lax.axis_index("s") * _NC + lax.axis_index("c")
    ubase = wid * upw
    pltpu.sync_copy(uid_hbm.at[pl.ds(ubase, upw)], uidb)
    pltpu.async_copy(umlp_hbm.at[uidb], urows, semu).wait()
    pltpu.sync_copy(urows, uemlp_out.at[pl.ds(ubase, upw)])
    pltpu.async_copy(umf_hbm.at[uidb], urows, semu).wait()
    pltpu.sync_copy(urows, uemf_out.at[pl.ds(ubase, upw)])


def _sc_user_gather(uid, user_mlp, user_mf):
    b = uid.shape[0]
    upw = b // _NW
    mesh = plsc.VectorSubcoreMesh(core_axis_name="c", subcore_axis_name="s")
    return pl.kernel(
        functools.partial(_sc_user_body, upw),
        out_type=[
            jax.ShapeDtypeStruct((b, E), jnp.float32),
            jax.ShapeDtypeStruct((b, E), jnp.float32),
        ],
        mesh=mesh,
        scratch_types=[
            pltpu.VMEM((upw,), jnp.int32),
            pltpu.VMEM((upw, E), jnp.float32),
            pltpu.SemaphoreType.DMA,
        ],
    )(uid, user_mlp, user_mf)


def _sc_token_body(tok, nch,
                   idx_hbm, c_hbm, g_out,
                   idxall, rows0, rows1,
                   semg0, semg1, sems0, sems1):
    wid = lax.axis_index("s") * _NC + lax.axis_index("c")
    tbase = wid * (tok // _NW)
    pltpu.sync_copy(idx_hbm.at[wid], idxall)
    bufs = ((rows0, semg0, sems0), (rows1, semg1, sems1))

    def g_copy(c, rbuf, sem):
        return pltpu.make_async_copy(c_hbm.at[idxall.at[c]], rbuf, sem)

    def s_copy(c, rbuf, sem):
        dst = g_out.at[pl.ds(tbase + c * _CH, _CH)]
        return pltpu.make_async_copy(rbuf, dst, sem)

    g_copy(0, rows0, semg0).start()

    def body(i, carry):
        cbase = i * 2
        for bsel in range(2):
            c = cbase + bsel
            rcur, gcur, scur = bufs[bsel]
            rnxt, gnxt, snxt = bufs[1 - bsel]

            @pl.when(c + 1 < nch)
            def _():
                @pl.when(c >= 1)
                def _():
                    s_copy(c - 1, rnxt, snxt).wait()

                g_copy(c + 1, rnxt, gnxt).start()

            g_copy(c, rcur, gcur).wait()
            s_copy(c, rcur, scur).start()
        return carry

    lax.fori_loop(0, nch // 2, body, 0)
    if nch % 2 == 1:
        rcur, gcur, scur = bufs[(nch - 1) % 2]
        g_copy(nch - 1, rcur, gcur).wait()
        s_copy(nch - 1, rcur, scur).start()
    s_copy(nch - 2, bufs[(nch - 2) % 2][0], bufs[(nch - 2) % 2][2]).wait()
    s_copy(nch - 1, bufs[(nch - 1) % 2][0], bufs[(nch - 1) % 2][2]).wait()


def _sc_token_gather(idx3, c):
    nw, nch, ch = idx3.shape
    tok = nw * nch * ch
    mesh = plsc.VectorSubcoreMesh(core_axis_name="c", subcore_axis_name="s")
    return pl.kernel(
        functools.partial(_sc_token_body, tok, nch),
        out_type=jax.ShapeDtypeStruct((tok, E), jnp.int32),
        mesh=mesh,
        scratch_types=[
            pltpu.VMEM((nch, _CH), jnp.int32),
            pltpu.VMEM((_CH, E), jnp.int32),
            pltpu.VMEM((_CH, E), jnp.int32),
            pltpu.SemaphoreType.DMA,
            pltpu.SemaphoreType.DMA,
            pltpu.SemaphoreType.DMA,
            pltpu.SemaphoreType.DMA,
        ],
    )(idx3, c)


def _mlp_body(tb, g_ref, ue_ref, uemf_ref, w0u_ref, w1_ref, b1_ref,
              w2_ref, b2_ref, woutr_ref, bout_ref, out_ref):
    f32 = jnp.float32
    per_u = tb // _UB
    rowu = lax.broadcasted_iota(jnp.int32, (tb, _UB), 0) // per_u
    colu = lax.broadcasted_iota(jnp.int32, (tb, _UB), 1)
    eb = (rowu == colu).astype(f32)

    a = jnp.dot(ue_ref[:], w0u_ref[:], preferred_element_type=f32)
    up = uemf_ref[:] * woutr_ref[1:2, :]
    a_tok = jnp.dot(eb, a, preferred_element_type=f32)
    up_tok = jnp.dot(eb, up, preferred_element_type=f32)

    gu = lax.bitcast_convert_type(g_ref[:], jnp.uint32)
    gt = lax.bitcast_convert_type(gu & jnp.uint32(0xFFFF0000), f32)
    gm = lax.bitcast_convert_type(gu << 16, f32)
    h = jnp.maximum(gt + a_tok, 0.0)
    h = jnp.maximum(
        jnp.dot(h.astype(jnp.bfloat16), w1_ref[:],
                preferred_element_type=f32) + b1_ref[:], 0.0)
    h = jnp.maximum(
        jnp.dot(h.astype(jnp.bfloat16), w2_ref[:],
                preferred_element_type=f32) + b2_ref[:], 0.0)
    dn = (((1,), (1,)), ((), ()))
    lh = lax.dot_general(woutr_ref[0:1, :], h, dn,
                         preferred_element_type=f32)
    ones_row = jnp.ones((1, E), f32)
    lmf = lax.dot_general(ones_row, gm * up_tok, dn,
                          preferred_element_type=f32)
    out_ref[:] = (lh + lmf + bout_ref[0, 0])[None]


def _mlp(g, uemlp, uemf, w0u, w1, b1, w2, b2, woutr, bout, nusers, ublk0):
    tok = g.shape[0]
    tb = tok // (nusers // _UB)
    grid = nusers // _UB
    return pl.pallas_call(
        functools.partial(_mlp_body, tb),
        grid=(grid,),
        in_specs=[
            pl.BlockSpec((tb, E), lambda i: (i, 0)),
            pl.BlockSpec((_UB, E), lambda i, u0=ublk0: (u0 + i, 0)),
            pl.BlockSpec((_UB, E), lambda i, u0=ublk0: (u0 + i, 0)),
            pl.BlockSpec((E, E), lambda i: (0, 0)),
            pl.BlockSpec((E, E), lambda i: (0, 0)),
            pl.BlockSpec((1, E), lambda i: (0, 0)),
            pl.BlockSpec((E, E), lambda i: (0, 0)),
            pl.BlockSpec((1, E), lambda i: (0, 0)),
            pl.BlockSpec((2, E), lambda i: (0, 0)),
            pl.BlockSpec((1, 1), lambda i: (0, 0)),
        ],
        out_specs=pl.BlockSpec((1, 1, tb), lambda i: (i, 0, 0)),
        out_shape=jax.ShapeDtypeStruct((grid, 1, tb), jnp.float32),
    )(g, uemlp, uemf, w0u, w1, b1, w2, b2, woutr, bout)


def kernel(uid, seq, pos, neg, nbr, nbr_iid, user_mlp, item_mlp, user_mf,
           item_mf, W0, b0, W1, b1, W2, b2, Wout, bout):
    del seq, nbr, nbr_iid
    b_sz, l_sz = pos.shape

    w0u = W0[:E, :]
    w0i = W0[E:, :]
    woutr = Wout.reshape(2, E)

    uemlp, uemf = _sc_user_gather(uid.astype(jnp.int32), user_mlp, user_mf)
    c = _build_c(item_mlp, item_mf, w0i, b0)

    nphase = 4
    tok = 2 * b_sz * l_sz
    tokp = tok // nphase
    nch = tokp // (_NW * _CH)
    all_idx = jnp.concatenate([pos, neg], axis=1).reshape(-1).astype(jnp.int32)
    idx4 = all_idx.reshape(nphase, _NW, nch, _CH)

    w1b = W1.astype(jnp.bfloat16)
    w2b = W2.astype(jnp.bfloat16)
    nusers_p = b_sz // nphase
    parts = []
    for p in range(nphase):
        g_p = _sc_token_gather(idx4[p], c)
        parts.append(_mlp(g_p, uemlp, uemf, w0u, w1b, b1.reshape(1, E),
                          w2b, b2.reshape(1, E), woutr, bout.reshape(1, 1),
                          nusers_p, p * (nusers_p // _UB)))

    logits = jnp.concatenate(parts, axis=0)
    out2 = logits.reshape(b_sz, 2 * l_sz)
    pos_logits = out2[:, :l_sz, None]
    neg_logits = out2[:, l_sz:, None]
    return (pos_logits, neg_logits)
